# Initial kernel scaffold; baseline (speedup 1.0000x reference)
import functools

import jax
import jax.numpy as jnp
from jax import lax
from jax.experimental import pallas as pl
from jax.experimental.pallas import tpu as pltpu
from jax.experimental.pallas import tpu_sc as plsc

B = 16384
DIMS = [8, 8, 8, 2, 4, 3, 4, 4, 4]
SMALL_VOCABS = [3, 8, 4, 16, 8, 16]
HIDDEN = 64

NC = 2
NS = 16
NW = NC * NS
BPW = B // NW
CHUNK = 128
NCHUNK = BPW // CHUNK


def _sc_gather(idx_big, t0, t1, t2):
    mesh = plsc.VectorSubcoreMesh(core_axis_name="c", subcore_axis_name="s")

    @functools.partial(
        pl.kernel,
        mesh=mesh,
        out_type=[jax.ShapeDtypeStruct((B, 8), jnp.float32) for _ in range(3)],
        scratch_types=[
            pltpu.VMEM((NCHUNK, CHUNK), jnp.int32),
            pltpu.VMEM((NCHUNK, CHUNK), jnp.int32),
            pltpu.VMEM((NCHUNK, CHUNK), jnp.int32),
            pltpu.VMEM((BPW, 8), jnp.float32),
            pltpu.VMEM((BPW, 8), jnp.float32),
            pltpu.VMEM((BPW, 8), jnp.float32),
            pltpu.SemaphoreType.DMA,
            pltpu.SemaphoreType.DMA,
            pltpu.SemaphoreType.DMA,
        ],
    )
    def k(idx_hbm, t0_hbm, t1_hbm, t2_hbm, o0, o1, o2,
          i0, i1, i2, r0, r1, r2, s0, s1, s2):
        wid = lax.axis_index("s") * NC + lax.axis_index("c")
        base = wid * BPW
        tabs = (t0_hbm, t1_hbm, t2_hbm)
        idxs = (i0, i1, i2)
        rows = (r0, r1, r2)
        sems = (s0, s1, s2)
        outs = (o0, o1, o2)
        for f in range(3):
            pltpu.sync_copy(idx_hbm.at[f, wid], idxs[f])
        handles = []
        for f in range(3):
            for j in range(NCHUNK):
                handles.append(pltpu.async_copy(
                    tabs[f].at[idxs[f].at[j]],
                    rows[f].at[pl.ds(j * CHUNK, CHUNK)],
                    sems[f]))
        for h in handles:
            h.wait()
        for f in range(3):
            pltpu.sync_copy(rows[f], outs[f].at[pl.ds(base, BPW)])

    return k(idx_big, t0, t1, t2)


def _tc_mlp_body(g0, g1, g2, idx_s, time_col,
                 s0, s1, s2, s3, s4, s5, w1t, b1, w2t, b2, out):
    small = (s0, s1, s2, s3, s4, s5)
    z = jnp.dot(g0[...], w1t[0:8, :], preferred_element_type=jnp.float32)
    z = z + jnp.dot(g1[...], w1t[8:16, :], preferred_element_type=jnp.float32)
    z = z + jnp.dot(g2[...], w1t[16:24, :], preferred_element_type=jnp.float32)
    off = 24
    for f in range(6):
        v = SMALL_VOCABS[f]
        d = DIMS[3 + f]
        proj = jnp.dot(small[f][...], w1t[off:off + d, :],
                       preferred_element_type=jnp.float32)
        ids = idx_s[:, f:f + 1]
        onehot = (ids == lax.broadcasted_iota(jnp.int32, (1, v), 1)
                  ).astype(jnp.float32)
        z = z + jnp.dot(onehot, proj, preferred_element_type=jnp.float32)
        off += d
    z = z + time_col[...] * w1t[45:46, :]
    z = z + b1[...]
    h1 = jnp.maximum(z, 0.0)
    o = jnp.dot(h1, w2t[...], preferred_element_type=jnp.float32) + b2[...]
    out[...] = jax.nn.sigmoid(o)


def kernel(x, W_emb0, W_emb1, W_emb2, W_emb3, W_emb4, W_emb5, W_emb6,
           W_emb7, W_emb8, W1, b1, W2, b2):
    idx = x[:, :9].astype(jnp.int32)
    idx_big = idx[:, :3].T.reshape(3, NW, NCHUNK, CHUNK)
    idx_small = idx[:, 3:9]
    time_col = x[:, 9:10]

    g0, g1, g2 = _sc_gather(idx_big, W_emb0, W_emb1, W_emb2)

    w1t = W1.T
    w2t = W2.T
    out = pl.pallas_call(
        _tc_mlp_body,
        out_shape=jax.ShapeDtypeStruct((B, 1), jnp.float32),
    )(g0, g1, g2, idx_small, time_col,
      W_emb3, W_emb4, W_emb5, W_emb6, W_emb7, W_emb8,
      w1t, b1.reshape(1, HIDDEN), w2t, b2.reshape(1, 1))
    return jnp.squeeze(out, axis=-1)


# speedup vs baseline: 1.1993x; 1.1993x over previous
import functools

import jax
import jax.numpy as jnp
from jax import lax
from jax.experimental import pallas as pl
from jax.experimental.pallas import tpu as pltpu
from jax.experimental.pallas import tpu_sc as plsc

B = 16384
DIMS = [8, 8, 8, 2, 4, 3, 4, 4, 4]
SMALL_VOCABS = [3, 8, 4, 16, 8, 16]
HIDDEN = 64

NC = 2
NS = 16
NW = NC * NS
BPW = B // NW
CHUNK = 128
NCHUNK = BPW // CHUNK


def _sc_gather(idx_big, t0, t1, t2):
    mesh = plsc.VectorSubcoreMesh(core_axis_name="c", subcore_axis_name="s")

    @functools.partial(
        pl.kernel,
        mesh=mesh,
        compiler_params=pltpu.CompilerParams(use_tc_tiling_on_sc=False),
        out_type=[jax.ShapeDtypeStruct((B, 8), jnp.float32) for _ in range(3)],
        scratch_types=[
            pltpu.VMEM((NCHUNK, CHUNK), jnp.int32),
            pltpu.VMEM((NCHUNK, CHUNK), jnp.int32),
            pltpu.VMEM((NCHUNK, CHUNK), jnp.int32),
            pltpu.VMEM((BPW, 8), jnp.float32),
            pltpu.VMEM((BPW, 8), jnp.float32),
            pltpu.VMEM((BPW, 8), jnp.float32),
            pltpu.SemaphoreType.DMA,
            pltpu.SemaphoreType.DMA,
            pltpu.SemaphoreType.DMA,
        ],
    )
    def k(idx_hbm, t0_hbm, t1_hbm, t2_hbm, o0, o1, o2,
          i0, i1, i2, r0, r1, r2, s0, s1, s2):
        wid = lax.axis_index("s") * NC + lax.axis_index("c")
        base = wid * BPW
        tabs = (t0_hbm, t1_hbm, t2_hbm)
        idxs = (i0, i1, i2)
        rows = (r0, r1, r2)
        sems = (s0, s1, s2)
        outs = (o0, o1, o2)
        for f in range(3):
            pltpu.sync_copy(idx_hbm.at[f, wid], idxs[f])
        handles = []
        for f in range(3):
            for j in range(NCHUNK):
                handles.append(pltpu.async_copy(
                    tabs[f].at[idxs[f].at[j]],
                    rows[f].at[pl.ds(j * CHUNK, CHUNK)],
                    sems[f]))
        for h in handles:
            h.wait()
        for f in range(3):
            pltpu.sync_copy(rows[f], outs[f].at[pl.ds(base, BPW)])

    return k(idx_big, t0, t1, t2)


def _tc_mlp_body(g0, g1, g2, idx_s, time_col,
                 s0, s1, s2, s3, s4, s5, w1t, b1, w2t, b2, out):
    small = (s0, s1, s2, s3, s4, s5)
    z = jnp.dot(g0[...], w1t[0:8, :], preferred_element_type=jnp.float32)
    z = z + jnp.dot(g1[...], w1t[8:16, :], preferred_element_type=jnp.float32)
    z = z + jnp.dot(g2[...], w1t[16:24, :], preferred_element_type=jnp.float32)
    off = 24
    for f in range(6):
        v = SMALL_VOCABS[f]
        d = DIMS[3 + f]
        proj = jnp.dot(small[f][...], w1t[off:off + d, :],
                       preferred_element_type=jnp.float32)
        ids = idx_s[:, f:f + 1]
        onehot = (ids == lax.broadcasted_iota(jnp.int32, (1, v), 1)
                  ).astype(jnp.float32)
        z = z + jnp.dot(onehot, proj, preferred_element_type=jnp.float32)
        off += d
    z = z + time_col[...] * w1t[45:46, :]
    z = z + b1[...]
    h1 = jnp.maximum(z, 0.0)
    o = jnp.dot(h1, w2t[...], preferred_element_type=jnp.float32) + b2[...]
    out[...] = jax.nn.sigmoid(o)


def kernel(x, W_emb0, W_emb1, W_emb2, W_emb3, W_emb4, W_emb5, W_emb6,
           W_emb7, W_emb8, W1, b1, W2, b2):
    idx = x[:, :9].astype(jnp.int32)
    idx_big = idx[:, :3].T.reshape(3, NW, NCHUNK, CHUNK)
    idx_small = idx[:, 3:9]
    time_col = x[:, 9:10]

    g0, g1, g2 = _sc_gather(idx_big, W_emb0, W_emb1, W_emb2)

    w1t = W1.T
    w2t = W2.T
    BB = 2048
    row_blk = lambda w: pl.BlockSpec((BB, w), lambda i: (i, 0))
    full = lambda s: pl.BlockSpec(s, lambda i: (0, 0))
    out = pl.pallas_call(
        _tc_mlp_body,
        grid=(B // BB,),
        in_specs=[row_blk(8), row_blk(8), row_blk(8), row_blk(6), row_blk(1),
                  full((3, 2)), full((8, 4)), full((4, 3)), full((16, 4)),
                  full((8, 4)), full((16, 4)),
                  full((46, HIDDEN)), full((1, HIDDEN)), full((HIDDEN, 1)),
                  full((1, 1))],
        out_specs=row_blk(1),
        out_shape=jax.ShapeDtypeStruct((B, 1), jnp.float32),
    )(g0, g1, g2, idx_small, time_col,
      W_emb3, W_emb4, W_emb5, W_emb6, W_emb7, W_emb8,
      w1t, b1.reshape(1, HIDDEN), w2t, b2.reshape(1, 1))
    return jnp.squeeze(out, axis=-1)


# speedup vs baseline: 1.2243x; 1.0208x over previous
import functools

import jax
import jax.numpy as jnp
from jax import lax
from jax.experimental import pallas as pl
from jax.experimental.pallas import tpu as pltpu
from jax.experimental.pallas import tpu_sc as plsc

B = 16384
DIMS = [8, 8, 8, 2, 4, 3, 4, 4, 4]
SMALL_VOCABS = [3, 8, 4, 16, 8, 16]
HIDDEN = 64

NC = 2
NS = 16
L = 16
NW = NC * NS
BPW = B // NW
CHUNK = 128
NCHUNK = BPW // CHUNK
NGRP = BPW // L


def _sc_gather(x, t0, t1, t2):
    mesh = plsc.VectorSubcoreMesh(core_axis_name="c", subcore_axis_name="s")

    @functools.partial(
        pl.kernel,
        mesh=mesh,
        compiler_params=pltpu.CompilerParams(use_tc_tiling_on_sc=False,
                                             needs_layout_passes=False),
        out_type=[jax.ShapeDtypeStruct((B, 8), jnp.float32) for _ in range(3)],
        scratch_types=[
            pltpu.VMEM((BPW, 10), jnp.float32),
            pltpu.VMEM((NCHUNK, CHUNK), jnp.int32),
            pltpu.VMEM((NCHUNK, CHUNK), jnp.int32),
            pltpu.VMEM((NCHUNK, CHUNK), jnp.int32),
            pltpu.VMEM((BPW, 8), jnp.float32),
            pltpu.VMEM((BPW, 8), jnp.float32),
            pltpu.VMEM((BPW, 8), jnp.float32),
            pltpu.SemaphoreType.DMA,
            pltpu.SemaphoreType.DMA,
            pltpu.SemaphoreType.DMA,
        ],
    )
    def k(x_hbm, t0_hbm, t1_hbm, t2_hbm, o0, o1, o2,
          xv, i0, i1, i2, r0, r1, r2, s0, s1, s2):
        wid = lax.axis_index("s") * NC + lax.axis_index("c")
        base = wid * BPW
        tabs = (t0_hbm, t1_hbm, t2_hbm)
        idxs = (i0, i1, i2)
        rows = (r0, r1, r2)
        sems = (s0, s1, s2)
        outs = (o0, o1, o2)

        pltpu.sync_copy(x_hbm.at[pl.ds(base, BPW)], xv)
        lane = lax.broadcasted_iota(jnp.int32, (L,), 0)
        for f in range(3):
            col = jnp.full((L,), f, jnp.int32)
            for g in range(NGRP):
                v = plsc.load_gather(xv, [g * L + lane, col])
                idxs[f].at[g // 8][pl.ds((g % 8) * L, L)] = v.astype(jnp.int32)

        handles = []
        for f in range(3):
            for j in range(NCHUNK):
                handles.append(pltpu.async_copy(
                    tabs[f].at[idxs[f].at[j]],
                    rows[f].at[pl.ds(j * CHUNK, CHUNK)],
                    sems[f]))
        for h in handles:
            h.wait()
        for f in range(3):
            pltpu.sync_copy(rows[f], outs[f].at[pl.ds(base, BPW)])

    return k(x, t0, t1, t2)


def _tc_mlp_body(g0, g1, g2, xr,
                 s0, s1, s2, s3, s4, s5, w1t, b1, w2t, b2, out):
    small = (s0, s1, s2, s3, s4, s5)
    z = jnp.dot(g0[...], w1t[0:8, :], preferred_element_type=jnp.float32)
    z = z + jnp.dot(g1[...], w1t[8:16, :], preferred_element_type=jnp.float32)
    z = z + jnp.dot(g2[...], w1t[16:24, :], preferred_element_type=jnp.float32)
    off = 24
    for f in range(6):
        v = SMALL_VOCABS[f]
        d = DIMS[3 + f]
        proj = jnp.dot(small[f][...], w1t[off:off + d, :],
                       preferred_element_type=jnp.float32)
        ids = xr[:, 3 + f:4 + f].astype(jnp.int32)
        onehot = (ids == lax.broadcasted_iota(jnp.int32, (1, v), 1)
                  ).astype(jnp.float32)
        z = z + jnp.dot(onehot, proj, preferred_element_type=jnp.float32)
        off += d
    z = z + xr[:, 9:10] * w1t[45:46, :]
    z = z + b1[...]
    h1 = jnp.maximum(z, 0.0)
    o = jnp.dot(h1, w2t[...], preferred_element_type=jnp.float32) + b2[...]
    out[...] = jax.nn.sigmoid(o)


def kernel(x, W_emb0, W_emb1, W_emb2, W_emb3, W_emb4, W_emb5, W_emb6,
           W_emb7, W_emb8, W1, b1, W2, b2):
    g0, g1, g2 = _sc_gather(x, W_emb0, W_emb1, W_emb2)

    w1t = W1.T
    w2t = W2.T
    BB = 2048
    row_blk = lambda w: pl.BlockSpec((BB, w), lambda i: (i, 0))
    full = lambda s: pl.BlockSpec(s, lambda i: (0, 0))
    out = pl.pallas_call(
        _tc_mlp_body,
        grid=(B // BB,),
        in_specs=[row_blk(8), row_blk(8), row_blk(8), row_blk(10),
                  full((3, 2)), full((8, 4)), full((4, 3)), full((16, 4)),
                  full((8, 4)), full((16, 4)),
                  full((46, HIDDEN)), full((1, HIDDEN)), full((HIDDEN, 1)),
                  full((1, 1))],
        out_specs=row_blk(1),
        out_shape=jax.ShapeDtypeStruct((B, 1), jnp.float32),
    )(g0, g1, g2, x,
      W_emb3, W_emb4, W_emb5, W_emb6, W_emb7, W_emb8,
      w1t, b1.reshape(1, HIDDEN), w2t, b2.reshape(1, 1))
    return jnp.squeeze(out, axis=-1)
